# baseline (speedup 1.0000x reference)
import functools

import jax
import jax.numpy as jnp
from jax import lax
from jax.experimental import pallas as pl
from jax.experimental.pallas import tpu as pltpu
from jax.experimental.pallas import tpu_sc as plsc

H = 1024
EPS = 1e-5
NC, NS = 2, 16
NW = NC * NS
CHUNK = 32
S_SPLITS = (512, 512, 1024, 1024, 1024)
RBS = (512, 512, 1024, 1024, 1024)


def _sc_gather(emb_table, ids_chunk, rows):
    b_per_w = rows // NW
    nch = b_per_w // CHUNK
    mesh = plsc.VectorSubcoreMesh(core_axis_name="c", subcore_axis_name="s")

    @functools.partial(
        pl.kernel,
        out_type=jax.ShapeDtypeStruct((rows, H), jnp.float32),
        mesh=mesh,
        scratch_types=[
            pltpu.VMEM((b_per_w,), jnp.int32),
            pltpu.VMEM((CHUNK, H), jnp.float32),
            pltpu.VMEM((CHUNK, H), jnp.float32),
            pltpu.SemaphoreType.DMA,
            pltpu.SemaphoreType.DMA,
            pltpu.SemaphoreType.DMA,
            pltpu.SemaphoreType.DMA,
        ],
    )
    def gather_k(table_hbm, idx_hbm, out_hbm, idx_v, r0, r1, g0, g1, w0, w1):
        wid = lax.axis_index("s") * NC + lax.axis_index("c")
        base = wid * b_per_w
        pltpu.sync_copy(idx_hbm.at[pl.ds(base, b_per_w)], idx_v)

        bufs = (r0, r1)
        gsems = (g0, g1)
        wsems = (w0, w1)
        gh = [None] * nch
        wh = [None] * nch
        gh[0] = pltpu.async_copy(
            table_hbm.at[idx_v.at[pl.ds(0, CHUNK)]], bufs[0], gsems[0])
        for c in range(1, nch):
            b = c % 2
            if c >= 2:
                wh[c - 2].wait()
            gh[c] = pltpu.async_copy(
                table_hbm.at[idx_v.at[pl.ds(c * CHUNK, CHUNK)]],
                bufs[b], gsems[b])
            gh[c - 1].wait()
            wh[c - 1] = pltpu.async_copy(
                bufs[1 - b], out_hbm.at[pl.ds(base + (c - 1) * CHUNK, CHUNK)],
                wsems[1 - b])
        last = nch - 1
        gh[last].wait()
        wh[last] = pltpu.async_copy(
            bufs[last % 2], out_hbm.at[pl.ds(base + last * CHUNK, CHUNK)],
            wsems[last % 2])
        if nch >= 2:
            wh[last - 1].wait()
        wh[last].wait()

    return gather_k(emb_table, ids_chunk)


def _ln_body_first(tok_ref, pos_ref, g_ref, b_ref, o_ref):
    x = tok_ref[...] + pos_ref[...]
    mean = jnp.mean(x, axis=1, keepdims=True)
    xc = x - mean
    var = jnp.mean(xc * xc, axis=1, keepdims=True)
    inv = lax.rsqrt(var + EPS)
    o_ref[...] = (xc * inv) * g_ref[...] + b_ref[...]


def _ln_body_chained(acc_ref, tok_ref, pos_ref, g_ref, b_ref, o_ref):
    del acc_ref
    _ln_body_first(tok_ref, pos_ref, g_ref, b_ref, o_ref)


def _tc_layernorm_chunk(acc, tok, pos, gamma, beta, b, s, s_c, off, rb):
    nposb_c = s_c // rb
    sposb = s // rb
    offb = off // rb
    grid = (nposb_c, b)
    tok_spec = pl.BlockSpec((rb, H), lambda i, j: (j * nposb_c + i, 0))
    pos_spec = pl.BlockSpec((rb, H), lambda i, j: (offb + i, 0))
    vec_spec = pl.BlockSpec((1, H), lambda i, j: (0, 0))
    out_spec = pl.BlockSpec((rb, H), lambda i, j: (j * sposb + offb + i, 0))
    out_shape = jax.ShapeDtypeStruct((b * s, H), jnp.float32)
    if acc is None:
        return pl.pallas_call(
            _ln_body_first,
            grid=grid,
            in_specs=[tok_spec, pos_spec, vec_spec, vec_spec],
            out_specs=out_spec,
            out_shape=out_shape,
        )(tok, pos, gamma, beta)
    acc_spec = pl.BlockSpec((8, 128), lambda i, j: (0, 0))
    return pl.pallas_call(
        _ln_body_chained,
        grid=grid,
        in_specs=[acc_spec, tok_spec, pos_spec, vec_spec, vec_spec],
        out_specs=out_spec,
        out_shape=out_shape,
        input_output_aliases={0: 0},
    )(acc, tok, pos, gamma, beta)


def kernel(input_ids, emb_table, pos_table, gamma, beta):
    b, s = input_ids.shape
    ids = input_ids.astype(jnp.int32)
    g2 = gamma.reshape(1, H)
    b2 = beta.reshape(1, H)

    offs = [0]
    for s_c in S_SPLITS:
        offs.append(offs[-1] + s_c)
    assert offs[-1] == s

    toks = [
        _sc_gather(
            emb_table,
            lax.slice(ids, (0, offs[k]), (b, offs[k + 1])).reshape(b * s_c),
            b * s_c,
        )
        for k, s_c in enumerate(S_SPLITS)
    ]
    acc = None
    for k, s_c in enumerate(S_SPLITS):
        acc = _tc_layernorm_chunk(
            acc, toks[k], pos_table, g2, b2, b, s, s_c, offs[k], RBS[k])
    return acc.reshape(b, s, H)


# speedup vs baseline: 1.0220x; 1.0220x over previous
import functools

import jax
import jax.numpy as jnp
from jax import lax
from jax.experimental import pallas as pl
from jax.experimental.pallas import tpu as pltpu
from jax.experimental.pallas import tpu_sc as plsc

H = 1024
EPS = 1e-5
NC, NS = 2, 16
NW = NC * NS
CHUNK = 32
S_SPLITS = (1024, 1024, 1024, 1024)
RBS = (1024, 1024, 1024, 1024)


def _sc_gather(emb_table, ids_flat, s_full, off, s_c, b):
    rows = b * s_c
    b_per_w = rows // NW
    nch = b_per_w // CHUNK
    assert s_c % b_per_w == 0
    mesh = plsc.VectorSubcoreMesh(core_axis_name="c", subcore_axis_name="s")

    @functools.partial(
        pl.kernel,
        out_type=jax.ShapeDtypeStruct((rows, H), jnp.float32),
        mesh=mesh,
        scratch_types=[
            pltpu.VMEM((b_per_w,), jnp.int32),
            pltpu.VMEM((CHUNK, H), jnp.float32),
            pltpu.VMEM((CHUNK, H), jnp.float32),
            pltpu.SemaphoreType.DMA,
            pltpu.SemaphoreType.DMA,
            pltpu.SemaphoreType.DMA,
            pltpu.SemaphoreType.DMA,
        ],
    )
    def gather_k(table_hbm, idx_hbm, out_hbm, idx_v, r0, r1, g0, g1, w0, w1):
        wid = lax.axis_index("s") * NC + lax.axis_index("c")
        base = wid * b_per_w
        src = (base // s_c) * s_full + off + base % s_c
        pltpu.sync_copy(idx_hbm.at[pl.ds(src, b_per_w)], idx_v)

        bufs = (r0, r1)
        gsems = (g0, g1)
        wsems = (w0, w1)
        gh = [None] * nch
        wh = [None] * nch
        gh[0] = pltpu.async_copy(
            table_hbm.at[idx_v.at[pl.ds(0, CHUNK)]], bufs[0], gsems[0])
        for c in range(1, nch):
            b = c % 2
            if c >= 2:
                wh[c - 2].wait()
            gh[c] = pltpu.async_copy(
                table_hbm.at[idx_v.at[pl.ds(c * CHUNK, CHUNK)]],
                bufs[b], gsems[b])
            gh[c - 1].wait()
            wh[c - 1] = pltpu.async_copy(
                bufs[1 - b], out_hbm.at[pl.ds(base + (c - 1) * CHUNK, CHUNK)],
                wsems[1 - b])
        last = nch - 1
        gh[last].wait()
        wh[last] = pltpu.async_copy(
            bufs[last % 2], out_hbm.at[pl.ds(base + last * CHUNK, CHUNK)],
            wsems[last % 2])
        if nch >= 2:
            wh[last - 1].wait()
        wh[last].wait()

    return gather_k(emb_table, ids_flat)


def _ln_body_first(tok_ref, pos_ref, g_ref, b_ref, o_ref):
    x = tok_ref[...] + pos_ref[...]
    mean = jnp.mean(x, axis=1, keepdims=True)
    xc = x - mean
    var = jnp.mean(xc * xc, axis=1, keepdims=True)
    inv = lax.rsqrt(var + EPS)
    o_ref[...] = (xc * inv) * g_ref[...] + b_ref[...]


def _ln_body_chained(acc_ref, tok_ref, pos_ref, g_ref, b_ref, o_ref):
    del acc_ref
    _ln_body_first(tok_ref, pos_ref, g_ref, b_ref, o_ref)


def _tc_layernorm_chunk(acc, tok, pos, gamma, beta, b, s, s_c, off, rb):
    nposb_c = s_c // rb
    sposb = s // rb
    offb = off // rb
    grid = (nposb_c, b)
    tok_spec = pl.BlockSpec((rb, H), lambda i, j: (j * nposb_c + i, 0))
    pos_spec = pl.BlockSpec((rb, H), lambda i, j: (offb + i, 0))
    vec_spec = pl.BlockSpec((1, H), lambda i, j: (0, 0))
    out_spec = pl.BlockSpec((rb, H), lambda i, j: (j * sposb + offb + i, 0))
    out_shape = jax.ShapeDtypeStruct((b * s, H), jnp.float32)
    if acc is None:
        return pl.pallas_call(
            _ln_body_first,
            grid=grid,
            in_specs=[tok_spec, pos_spec, vec_spec, vec_spec],
            out_specs=out_spec,
            out_shape=out_shape,
        )(tok, pos, gamma, beta)
    acc_spec = pl.BlockSpec((8, 128), lambda i, j: (0, 0))
    return pl.pallas_call(
        _ln_body_chained,
        grid=grid,
        in_specs=[acc_spec, tok_spec, pos_spec, vec_spec, vec_spec],
        out_specs=out_spec,
        out_shape=out_shape,
        input_output_aliases={0: 0},
    )(acc, tok, pos, gamma, beta)


def kernel(input_ids, emb_table, pos_table, gamma, beta):
    b, s = input_ids.shape
    ids = input_ids.astype(jnp.int32)
    g2 = gamma.reshape(1, H)
    b2 = beta.reshape(1, H)

    offs = [0]
    for s_c in S_SPLITS:
        offs.append(offs[-1] + s_c)
    assert offs[-1] == s

    ids_flat = ids.reshape(b * s)
    toks = [
        _sc_gather(emb_table, ids_flat, s, offs[k], s_c, b)
        for k, s_c in enumerate(S_SPLITS)
    ]
    acc = None
    for k, s_c in enumerate(S_SPLITS):
        acc = _tc_layernorm_chunk(
            acc, toks[k], pos_table, g2, b2, b, s, s_c, offs[k], RBS[k])
    return acc.reshape(b, s, H)
